# baseline (speedup 1.0000x reference)
import functools

import jax
import jax.numpy as jnp
from jax import lax
from jax.experimental import pallas as pl
from jax.experimental.pallas import tpu as pltpu
from jax.experimental.pallas import tpu_sc as plsc

_MAXP = 128
_D = 64
_LQ = 2048
_LK = 2048
_TFLAT = 257 * _D

_NDT = 8
_NJT = _LK // 128
_NIQ = 4
_IB = _LQ // _NIQ
_SPAN = _LK + _IB
_NGROW = _SPAN // 128
_NG = 8 * _NGROW
_GDEPTH = 4
_RDEPTH = 2
_M = _IB // 8


def _rp_body(table_hbm, delta_hbm, out_hbm, idx_v, slab_v, delta_v, gsem, sem):
    wid = lax.axis_index("s") * 2 + lax.axis_index("c")
    dt = wid % _NDT
    iq = wid // _NDT
    d0 = dt * 8
    i0 = iq * _IB
    t_lo = (_LQ - _IB) - i0

    pltpu.sync_copy(delta_hbm, delta_v)
    delta = delta_v[...]
    base = t_lo - (_LQ - 1)

    def phase(s, carry):
        def fill_idx(kk, c2):
            t = lax.iota(jnp.int32, 16) + (kk * 16 + base + s) + delta
            t = jnp.minimum(jnp.maximum(t, -_MAXP), _MAXP) + _MAXP
            flat0 = t * _D + d0
            for dm in range(8):
                idx_v[dm * _NGROW + kk // 8,
                      pl.ds((kk % 8) * 16, 16)] = flat0 + dm
            return c2

        lax.fori_loop(0, _SPAN // 16, fill_idx, 0)

        def gcopy(g):
            return pltpu.make_async_copy(
                table_hbm.at[idx_v.at[g]],
                slab_v.at[g // _NGROW, pl.ds((g % _NGROW) * 128, 128)],
                gsem,
            )

        for g in range(_GDEPTH):
            gcopy(g).start()

        def gpump(g, c2):
            gcopy(g + _GDEPTH).start()
            gcopy(g).wait()
            return c2

        lax.fori_loop(0, _NG - _GDEPTH, gpump, 0)
        for g in range(_NG - _GDEPTH, _NG):
            gcopy(g).wait()

        def row_copies(m):
            ri = (7 - s) + 8 * m
            off = 8 * ((_M - 1) - m)
            return [
                pltpu.make_async_copy(
                    slab_v.at[:, pl.ds(off + c * 128, 128)],
                    out_hbm.at[i0 + ri, dt, c],
                    sem,
                )
                for c in range(_NJT)
            ]

        for k in range(_RDEPTH):
            for cp in row_copies(k):
                cp.start()

        def pump(m, c2):
            for cp in row_copies(m + _RDEPTH):
                cp.start()
            for cp in row_copies(m):
                cp.wait()
            return c2

        lax.fori_loop(0, _M - _RDEPTH, pump, 0)
        for k in range(_M - _RDEPTH, _M):
            for cp in row_copies(k):
                cp.wait()
        return carry

    lax.fori_loop(0, 8, phase, 0)


_rp_call = functools.partial(
    pl.kernel,
    mesh=plsc.VectorSubcoreMesh(core_axis_name="c", subcore_axis_name="s"),
    out_type=jax.ShapeDtypeStruct((_LQ, _NDT, _NJT, 8, 128), jnp.float32),
    scratch_types=[
        pltpu.VMEM((_NG, 128), jnp.int32),
        pltpu.VMEM((8, _SPAN), jnp.float32),
        pltpu.VMEM((16,), jnp.int32),
        pltpu.SemaphoreType.DMA,
        pltpu.SemaphoreType.DMA,
    ],
    compiler_params=pltpu.CompilerParams(use_tc_tiling_on_sc=False),
)(_rp_body)


def kernel(length_q, length_k, embeddings_table):
    tbl = embeddings_table.astype(jnp.float32).reshape(_TFLAT)
    delta = jnp.zeros((16,), jnp.int32) + (
        jnp.asarray(length_k, jnp.int32) - jnp.asarray(length_q, jnp.int32))
    out5 = _rp_call(tbl, delta)
    return out5.transpose(0, 2, 4, 1, 3).reshape(_LQ, _LK, _D)
